# baseline (speedup 1.0000x reference)
import jax
import jax.numpy as jnp
from jax import lax
from jax.experimental import pallas as pl
from jax.experimental.pallas import tpu as pltpu
from jax.experimental.pallas import tpu_sc as plsc

NC = 2
NS = 16
NW = NC * NS
L = 16

B = 16384
J = 20
EMB = 64

B_W = B // NW
CB = 32
NCH = B_W // CB
NEG_ROWS = CB * J
G = 128
NG = NEG_ROWS // G
NB_W = B_W * J
PAD = L

V = 1000000
CPB = 2048
UPB = CPB // 2
NFB = V // CPB
TAIL0 = NFB * CPB
TAILN = V - TAIL0
VU = (NFB + 1) * UPB


def _tx_body(t_ref, tail_ref, out_ref, scr, sem):
    j = pl.program_id(0)

    @pl.when(j < NFB)
    def _main():
        pltpu.make_async_copy(
            t_ref.at[:, pl.ds(j * CPB, CPB)], scr, sem).start()
        pltpu.make_async_copy(
            t_ref.at[:, pl.ds(j * CPB, CPB)], scr, sem).wait()
        out_ref[:, 0:EMB] = scr[:, 0:UPB].T
        out_ref[:, EMB:2 * EMB] = scr[:, UPB:CPB].T

    @pl.when(j == NFB)
    def _tail():
        out_ref[...] = tail_ref[...]


def _transpose_pack(table, tail):
    return pl.pallas_call(
        _tx_body,
        grid=(NFB + 1,),
        in_specs=[
            pl.BlockSpec(memory_space=pl.ANY),
            pl.BlockSpec((UPB, 2 * EMB), lambda j: (0, 0)),
        ],
        out_specs=pl.BlockSpec((UPB, 2 * EMB), lambda j: (j, 0)),
        out_shape=jax.ShapeDtypeStruct((VU, 2 * EMB), jnp.float32),
        scratch_shapes=[
            pltpu.VMEM((EMB, CPB), jnp.float32),
            pltpu.SemaphoreType.DMA,
        ],
    )(table.T, tail)


def _sgns_body(c_h, o_h, n_h, vemb, uemb, pos_h, negout_h,
               cidx, oidx, nidx, cg, og, ng, vrows, orows, nrows,
               posb, negb, sem):
    w = lax.axis_index("s") * NC + lax.axis_index("c")

    pltpu.sync_copy(c_h.at[w], cidx.at[pl.ds(0, B_W)])
    pltpu.sync_copy(o_h.at[w], oidx.at[pl.ds(0, B_W)])
    pltpu.sync_copy(n_h.at[w], nidx.at[pl.ds(0, NB_W)])

    lane = lax.iota(jnp.int32, L)
    last = lane == (L - 1)

    def unit_of(x):
        return lax.shift_left(lax.shift_right_logical(x, 11), 10) | (x & 1023)

    def shift_co(i, carry):
        cg[pl.ds(i * L, L)] = unit_of(cidx[pl.ds(i * L, L)])
        og[pl.ds(i * L, L)] = unit_of(oidx[pl.ds(i * L, L)])
        return carry

    def shift_n(i, carry):
        ng[pl.ds(i * L, L)] = unit_of(nidx[pl.ds(i * L, L)])
        return carry

    lax.fori_loop(0, B_W // L, shift_co, 0)
    lax.fori_loop(0, NB_W // L, shift_n, 0)

    def chunk(ch, carry):
        cps = [
            pltpu.async_copy(vemb.at[cg.at[pl.ds(ch * CB, CB)]], vrows, sem),
            pltpu.async_copy(uemb.at[og.at[pl.ds(ch * CB, CB)]], orows, sem),
        ]
        for k in range(NG):
            cps.append(pltpu.async_copy(
                uemb.at[ng.at[pl.ds((ch * NG + k) * G, G)]],
                nrows.at[pl.ds(k * G, G)], sem))
        for cp in cps:
            cp.wait()

        def bbody(bl, c2):
            fb = ch * CB + bl
            co_vec = cidx[pl.ds(fb, L)]
            oo_vec = oidx[pl.ds(fb, L)]
            voff = jnp.where((co_vec[0] & 1024) != 0, EMB, 0)
            v0 = vrows[bl, pl.ds(voff, L)]
            v1 = vrows[bl, pl.ds(voff + L, L)]
            v2 = vrows[bl, pl.ds(voff + 2 * L, L)]
            v3 = vrows[bl, pl.ds(voff + 3 * L, L)]

            def dot_store(rref, row, off, pos):
                acc = rref[row, pl.ds(off, L)] * v0
                acc = acc + rref[row, pl.ds(off + L, L)] * v1
                acc = acc + rref[row, pl.ds(off + 2 * L, L)] * v2
                acc = acc + rref[row, pl.ds(off + 3 * L, L)] * v3
                s = plsc.cumsum(acc)
                idx = jnp.full((L,), pos, dtype=jnp.int32)
                plsc.store_scatter(negb, [idx], s, mask=last)

            po = jnp.where((oo_vec[0] & 1024) != 0, EMB, 0)
            acc = orows[bl, pl.ds(po, L)] * v0
            acc = acc + orows[bl, pl.ds(po + L, L)] * v1
            acc = acc + orows[bl, pl.ds(po + 2 * L, L)] * v2
            acc = acc + orows[bl, pl.ds(po + 3 * L, L)] * v3
            s = plsc.cumsum(acc)
            idx = jnp.full((L,), fb, dtype=jnp.int32)
            plsc.store_scatter(posb, [idx], s, mask=last)

            fnb = fb * J
            pv0 = nidx[pl.ds(fnb, L)]
            pv1 = nidx[pl.ds(fnb + 4, L)]
            for j in range(J):
                n_orig = pv0[j] if j < L else pv1[j - 4]
                dot_store(nrows, bl * J + j,
                          jnp.where((n_orig & 1024) != 0, EMB, 0), fnb + j)
            return c2

        lax.fori_loop(0, CB, bbody, 0)
        return carry

    lax.fori_loop(0, NCH, chunk, 0)

    def sig_pos(i, c2):
        x = posb[pl.ds(i * L, L)]
        posb[pl.ds(i * L, L)] = 1.0 / (1.0 + jnp.exp(-x))
        return c2

    def sig_neg(i, c2):
        x = negb[pl.ds(i * L, L)]
        negb[pl.ds(i * L, L)] = 1.0 / (1.0 + jnp.exp(x))
        return c2

    lax.fori_loop(0, B_W // L, sig_pos, 0)
    lax.fori_loop(0, NB_W // L, sig_neg, 0)

    pltpu.sync_copy(posb, pos_h.at[w])
    pltpu.sync_copy(negb, negout_h.at[w])


@jax.jit
def _sgns(c_h, o_h, n_h, vemb, uemb):
    mesh = plsc.VectorSubcoreMesh(core_axis_name="c", subcore_axis_name="s",
                                  num_cores=NC, num_subcores=NS)
    f = pl.kernel(
        _sgns_body,
        out_type=(
            jax.ShapeDtypeStruct((NW, B_W), jnp.float32),
            jax.ShapeDtypeStruct((NW, NB_W), jnp.float32),
        ),
        mesh=mesh,
        scratch_types=[
            pltpu.VMEM((B_W + PAD,), jnp.int32),
            pltpu.VMEM((B_W + PAD,), jnp.int32),
            pltpu.VMEM((NB_W + PAD,), jnp.int32),
            pltpu.VMEM((B_W,), jnp.int32),
            pltpu.VMEM((B_W,), jnp.int32),
            pltpu.VMEM((NB_W,), jnp.int32),
            pltpu.VMEM((CB, 2 * EMB), jnp.float32),
            pltpu.VMEM((CB, 2 * EMB), jnp.float32),
            pltpu.VMEM((NEG_ROWS, 2 * EMB), jnp.float32),
            pltpu.VMEM((B_W,), jnp.float32),
            pltpu.VMEM((NB_W,), jnp.float32),
            pltpu.SemaphoreType.DMA,
        ],
        compiler_params=pltpu.CompilerParams(needs_layout_passes=False,
                                             use_tc_tiling_on_sc=True),
    )
    return f(c_h, o_h, n_h, vemb, uemb)


def kernel(c, o, neg, vEmbedding, uEmbedding):
    c_h = c.reshape(NW, B_W).astype(jnp.int32)
    o_h = o.reshape(NW, B_W).astype(jnp.int32)
    n_h = neg.reshape(NW, NB_W).astype(jnp.int32)
    vtail = jnp.pad(vEmbedding[TAIL0:], ((0, UPB - TAILN), (0, EMB)))
    utail = jnp.pad(uEmbedding[TAIL0:], ((0, UPB - TAILN), (0, EMB)))
    v2 = _transpose_pack(vEmbedding, vtail)
    u2 = _transpose_pack(uEmbedding, utail)
    pos, negout = _sgns(c_h, o_h, n_h, v2, u2)
    return pos.reshape(B, 1), negout.reshape(B, J, 1)


# speedup vs baseline: 1.6866x; 1.6866x over previous
import jax
import jax.numpy as jnp
from jax import lax
from jax.experimental import pallas as pl
from jax.experimental.pallas import tpu as pltpu
from jax.experimental.pallas import tpu_sc as plsc

NC = 2
NS = 16
NW = NC * NS
L = 16

B = 16384
J = 20
EMB = 64

B_W = B // NW
CB = 32
NCH = B_W // CB
NEG_ROWS = CB * J
G = 128
NG = NEG_ROWS // G
NB_W = B_W * J
PAD = L

V = 1000000
CPB = 2048
UPB = CPB // 2
NFB = V // CPB
TAIL0 = NFB * CPB
TAILN = V - TAIL0
VU = (NFB + 1) * UPB


def _tx_body(t_ref, out_ref):
    eye = jnp.eye(EMB, dtype=jnp.float32)
    x = t_ref[...]
    out_ref[:, 0:EMB] = jax.lax.dot_general(
        x[:, 0:UPB], eye, (((0,), (0,)), ((), ())),
        preferred_element_type=jnp.float32)
    out_ref[:, EMB:2 * EMB] = jax.lax.dot_general(
        x[:, UPB:CPB], eye, (((0,), (0,)), ((), ())),
        preferred_element_type=jnp.float32)


def _transpose_pack(table):
    return pl.pallas_call(
        _tx_body,
        grid=(NFB + 1,),
        in_specs=[pl.BlockSpec((EMB, CPB), lambda j: (0, j))],
        out_specs=pl.BlockSpec((UPB, 2 * EMB), lambda j: (j, 0)),
        out_shape=jax.ShapeDtypeStruct((VU, 2 * EMB), jnp.float32),
    )(table.T)


def _sgns_body(c_h, o_h, n_h, vemb, uemb, pos_h, negout_h,
               cidx, oidx, nidx, cg, og, ng, vrows, orows, nrows,
               posb, negb, sem):
    w = lax.axis_index("s") * NC + lax.axis_index("c")

    pltpu.sync_copy(c_h.at[w], cidx.at[pl.ds(0, B_W)])
    pltpu.sync_copy(o_h.at[w], oidx.at[pl.ds(0, B_W)])
    pltpu.sync_copy(n_h.at[w], nidx.at[pl.ds(0, NB_W)])

    lane = lax.iota(jnp.int32, L)
    last = lane == (L - 1)

    def unit_of(x):
        return lax.shift_left(lax.shift_right_logical(x, 11), 10) | (x & 1023)

    def shift_co(i, carry):
        cg[pl.ds(i * L, L)] = unit_of(cidx[pl.ds(i * L, L)])
        og[pl.ds(i * L, L)] = unit_of(oidx[pl.ds(i * L, L)])
        return carry

    def shift_n(i, carry):
        ng[pl.ds(i * L, L)] = unit_of(nidx[pl.ds(i * L, L)])
        return carry

    lax.fori_loop(0, B_W // L, shift_co, 0)
    lax.fori_loop(0, NB_W // L, shift_n, 0)

    def chunk(ch, carry):
        cps = [
            pltpu.async_copy(vemb.at[cg.at[pl.ds(ch * CB, CB)]], vrows, sem),
            pltpu.async_copy(uemb.at[og.at[pl.ds(ch * CB, CB)]], orows, sem),
        ]
        for k in range(NG):
            cps.append(pltpu.async_copy(
                uemb.at[ng.at[pl.ds((ch * NG + k) * G, G)]],
                nrows.at[pl.ds(k * G, G)], sem))
        for cp in cps:
            cp.wait()

        def bbody(bl, c2):
            fb = ch * CB + bl
            co_vec = cidx[pl.ds(fb, L)]
            oo_vec = oidx[pl.ds(fb, L)]
            voff = jnp.where((co_vec[0] & 1024) != 0, EMB, 0)
            v0 = vrows[bl, pl.ds(voff, L)]
            v1 = vrows[bl, pl.ds(voff + L, L)]
            v2 = vrows[bl, pl.ds(voff + 2 * L, L)]
            v3 = vrows[bl, pl.ds(voff + 3 * L, L)]

            def dot_store(rref, row, off, pos):
                acc = rref[row, pl.ds(off, L)] * v0
                acc = acc + rref[row, pl.ds(off + L, L)] * v1
                acc = acc + rref[row, pl.ds(off + 2 * L, L)] * v2
                acc = acc + rref[row, pl.ds(off + 3 * L, L)] * v3
                s = plsc.cumsum(acc)
                idx = jnp.full((L,), pos, dtype=jnp.int32)
                plsc.store_scatter(negb, [idx], s, mask=last)

            po = jnp.where((oo_vec[0] & 1024) != 0, EMB, 0)
            acc = orows[bl, pl.ds(po, L)] * v0
            acc = acc + orows[bl, pl.ds(po + L, L)] * v1
            acc = acc + orows[bl, pl.ds(po + 2 * L, L)] * v2
            acc = acc + orows[bl, pl.ds(po + 3 * L, L)] * v3
            s = plsc.cumsum(acc)
            idx = jnp.full((L,), fb, dtype=jnp.int32)
            plsc.store_scatter(posb, [idx], s, mask=last)

            fnb = fb * J
            pv0 = nidx[pl.ds(fnb, L)]
            pv1 = nidx[pl.ds(fnb + 4, L)]
            for j in range(J):
                n_orig = pv0[j] if j < L else pv1[j - 4]
                dot_store(nrows, bl * J + j,
                          jnp.where((n_orig & 1024) != 0, EMB, 0), fnb + j)
            return c2

        lax.fori_loop(0, CB, bbody, 0)
        return carry

    lax.fori_loop(0, NCH, chunk, 0)

    def sig_pos(i, c2):
        x = posb[pl.ds(i * L, L)]
        posb[pl.ds(i * L, L)] = 1.0 / (1.0 + jnp.exp(-x))
        return c2

    def sig_neg(i, c2):
        x = negb[pl.ds(i * L, L)]
        negb[pl.ds(i * L, L)] = 1.0 / (1.0 + jnp.exp(x))
        return c2

    lax.fori_loop(0, B_W // L, sig_pos, 0)
    lax.fori_loop(0, NB_W // L, sig_neg, 0)

    pltpu.sync_copy(posb, pos_h.at[w])
    pltpu.sync_copy(negb, negout_h.at[w])


@jax.jit
def _sgns(c_h, o_h, n_h, vemb, uemb):
    mesh = plsc.VectorSubcoreMesh(core_axis_name="c", subcore_axis_name="s",
                                  num_cores=NC, num_subcores=NS)
    f = pl.kernel(
        _sgns_body,
        out_type=(
            jax.ShapeDtypeStruct((NW, B_W), jnp.float32),
            jax.ShapeDtypeStruct((NW, NB_W), jnp.float32),
        ),
        mesh=mesh,
        scratch_types=[
            pltpu.VMEM((B_W + PAD,), jnp.int32),
            pltpu.VMEM((B_W + PAD,), jnp.int32),
            pltpu.VMEM((NB_W + PAD,), jnp.int32),
            pltpu.VMEM((B_W,), jnp.int32),
            pltpu.VMEM((B_W,), jnp.int32),
            pltpu.VMEM((NB_W,), jnp.int32),
            pltpu.VMEM((CB, 2 * EMB), jnp.float32),
            pltpu.VMEM((CB, 2 * EMB), jnp.float32),
            pltpu.VMEM((NEG_ROWS, 2 * EMB), jnp.float32),
            pltpu.VMEM((B_W,), jnp.float32),
            pltpu.VMEM((NB_W,), jnp.float32),
            pltpu.SemaphoreType.DMA,
        ],
        compiler_params=pltpu.CompilerParams(needs_layout_passes=False,
                                             use_tc_tiling_on_sc=True),
    )
    return f(c_h, o_h, n_h, vemb, uemb)


def kernel(c, o, neg, vEmbedding, uEmbedding):
    c_h = c.reshape(NW, B_W).astype(jnp.int32)
    o_h = o.reshape(NW, B_W).astype(jnp.int32)
    n_h = neg.reshape(NW, NB_W).astype(jnp.int32)
    v2 = _transpose_pack(vEmbedding)
    u2 = _transpose_pack(uEmbedding)
    pos, negout = _sgns(c_h, o_h, n_h, v2, u2)
    return pos.reshape(B, 1), negout.reshape(B, J, 1)


# speedup vs baseline: 2.1508x; 1.2753x over previous
import jax
import jax.numpy as jnp
from jax import lax
from jax.experimental import pallas as pl
from jax.experimental.pallas import tpu as pltpu
from jax.experimental.pallas import tpu_sc as plsc

NC = 2
NS = 16
NW = NC * NS
L = 16

B = 16384
J = 20
EMB = 64

B_W = B // NW
CB = 32
NCH = B_W // CB
NEG_ROWS = CB * J
G = 128
NG = NEG_ROWS // G
NB_W = B_W * J
PAD = L

V = 1000000
CPB = 4096
UPB = CPB // 2
NFB = V // CPB
TAIL0 = NFB * CPB
TAILN = V - TAIL0
VU = (NFB + 1) * UPB
CPB_LOG2 = CPB.bit_length() - 1
UPB_LOG2 = UPB.bit_length() - 1


def _tx_body(t_ref, out_ref):
    eye = jnp.eye(EMB, dtype=jnp.float32)
    x = t_ref[...]
    out_ref[:, 0:EMB] = jax.lax.dot_general(
        x[:, 0:UPB], eye, (((0,), (0,)), ((), ())),
        preferred_element_type=jnp.float32)
    out_ref[:, EMB:2 * EMB] = jax.lax.dot_general(
        x[:, UPB:CPB], eye, (((0,), (0,)), ((), ())),
        preferred_element_type=jnp.float32)


def _transpose_pack(table):
    return pl.pallas_call(
        _tx_body,
        grid=(NFB + 1,),
        in_specs=[pl.BlockSpec((EMB, CPB), lambda j: (0, j))],
        out_specs=pl.BlockSpec((UPB, 2 * EMB), lambda j: (j, 0)),
        out_shape=jax.ShapeDtypeStruct((VU, 2 * EMB), jnp.float32),
    )(table.T)


def _sgns_body(c_h, o_h, n_h, vemb, uemb, pos_h, negout_h,
               cidx, oidx, nidx, cg, og, ng, vrows, orows, nrows,
               posb, negb, sem):
    w = lax.axis_index("s") * NC + lax.axis_index("c")

    pltpu.sync_copy(c_h.at[w], cidx.at[pl.ds(0, B_W)])
    pltpu.sync_copy(o_h.at[w], oidx.at[pl.ds(0, B_W)])
    pltpu.sync_copy(n_h.at[w], nidx.at[pl.ds(0, NB_W)])

    lane = lax.iota(jnp.int32, L)
    last = lane == (L - 1)

    def unit_of(x):
        return lax.shift_left(lax.shift_right_logical(x, CPB_LOG2),
                              UPB_LOG2) | (x & (UPB - 1))

    def shift_co(i, carry):
        cg[pl.ds(i * L, L)] = unit_of(cidx[pl.ds(i * L, L)])
        og[pl.ds(i * L, L)] = unit_of(oidx[pl.ds(i * L, L)])
        return carry

    def shift_n(i, carry):
        ng[pl.ds(i * L, L)] = unit_of(nidx[pl.ds(i * L, L)])
        return carry

    lax.fori_loop(0, B_W // L, shift_co, 0)
    lax.fori_loop(0, NB_W // L, shift_n, 0)

    def chunk(ch, carry):
        cps = [
            pltpu.async_copy(vemb.at[cg.at[pl.ds(ch * CB, CB)]], vrows, sem),
            pltpu.async_copy(uemb.at[og.at[pl.ds(ch * CB, CB)]], orows, sem),
        ]
        for k in range(NG):
            cps.append(pltpu.async_copy(
                uemb.at[ng.at[pl.ds((ch * NG + k) * G, G)]],
                nrows.at[pl.ds(k * G, G)], sem))
        for cp in cps:
            cp.wait()

        def bbody(bl, c2):
            fb = ch * CB + bl
            co_vec = cidx[pl.ds(fb, L)]
            oo_vec = oidx[pl.ds(fb, L)]
            voff = jnp.where((co_vec[0] & UPB) != 0, EMB, 0)
            v0 = vrows[bl, pl.ds(voff, L)]
            v1 = vrows[bl, pl.ds(voff + L, L)]
            v2 = vrows[bl, pl.ds(voff + 2 * L, L)]
            v3 = vrows[bl, pl.ds(voff + 3 * L, L)]

            def dot_store(rref, row, off, pos):
                acc = rref[row, pl.ds(off, L)] * v0
                acc = acc + rref[row, pl.ds(off + L, L)] * v1
                acc = acc + rref[row, pl.ds(off + 2 * L, L)] * v2
                acc = acc + rref[row, pl.ds(off + 3 * L, L)] * v3
                s = plsc.cumsum(acc)
                idx = jnp.full((L,), pos, dtype=jnp.int32)
                plsc.store_scatter(negb, [idx], s, mask=last)

            po = jnp.where((oo_vec[0] & UPB) != 0, EMB, 0)
            acc = orows[bl, pl.ds(po, L)] * v0
            acc = acc + orows[bl, pl.ds(po + L, L)] * v1
            acc = acc + orows[bl, pl.ds(po + 2 * L, L)] * v2
            acc = acc + orows[bl, pl.ds(po + 3 * L, L)] * v3
            s = plsc.cumsum(acc)
            idx = jnp.full((L,), fb, dtype=jnp.int32)
            plsc.store_scatter(posb, [idx], s, mask=last)

            fnb = fb * J
            pv0 = nidx[pl.ds(fnb, L)]
            pv1 = nidx[pl.ds(fnb + 4, L)]
            for j in range(J):
                n_orig = pv0[j] if j < L else pv1[j - 4]
                dot_store(nrows, bl * J + j,
                          jnp.where((n_orig & UPB) != 0, EMB, 0), fnb + j)
            return c2

        lax.fori_loop(0, CB, bbody, 0)
        return carry

    lax.fori_loop(0, NCH, chunk, 0)

    def sig_pos(i, c2):
        x = posb[pl.ds(i * L, L)]
        posb[pl.ds(i * L, L)] = 1.0 / (1.0 + jnp.exp(-x))
        return c2

    def sig_neg(i, c2):
        x = negb[pl.ds(i * L, L)]
        negb[pl.ds(i * L, L)] = 1.0 / (1.0 + jnp.exp(x))
        return c2

    lax.fori_loop(0, B_W // L, sig_pos, 0)
    lax.fori_loop(0, NB_W // L, sig_neg, 0)

    pltpu.sync_copy(posb, pos_h.at[w])
    pltpu.sync_copy(negb, negout_h.at[w])


@jax.jit
def _sgns(c_h, o_h, n_h, vemb, uemb):
    mesh = plsc.VectorSubcoreMesh(core_axis_name="c", subcore_axis_name="s",
                                  num_cores=NC, num_subcores=NS)
    f = pl.kernel(
        _sgns_body,
        out_type=(
            jax.ShapeDtypeStruct((NW, B_W), jnp.float32),
            jax.ShapeDtypeStruct((NW, NB_W), jnp.float32),
        ),
        mesh=mesh,
        scratch_types=[
            pltpu.VMEM((B_W + PAD,), jnp.int32),
            pltpu.VMEM((B_W + PAD,), jnp.int32),
            pltpu.VMEM((NB_W + PAD,), jnp.int32),
            pltpu.VMEM((B_W,), jnp.int32),
            pltpu.VMEM((B_W,), jnp.int32),
            pltpu.VMEM((NB_W,), jnp.int32),
            pltpu.VMEM((CB, 2 * EMB), jnp.float32),
            pltpu.VMEM((CB, 2 * EMB), jnp.float32),
            pltpu.VMEM((NEG_ROWS, 2 * EMB), jnp.float32),
            pltpu.VMEM((B_W,), jnp.float32),
            pltpu.VMEM((NB_W,), jnp.float32),
            pltpu.SemaphoreType.DMA,
        ],
        compiler_params=pltpu.CompilerParams(needs_layout_passes=False,
                                             use_tc_tiling_on_sc=True),
    )
    return f(c_h, o_h, n_h, vemb, uemb)


def kernel(c, o, neg, vEmbedding, uEmbedding):
    c_h = c.reshape(NW, B_W).astype(jnp.int32)
    o_h = o.reshape(NW, B_W).astype(jnp.int32)
    n_h = neg.reshape(NW, NB_W).astype(jnp.int32)
    v2 = _transpose_pack(vEmbedding)
    u2 = _transpose_pack(uEmbedding)
    pos, negout = _sgns(c_h, o_h, n_h, v2, u2)
    return pos.reshape(B, 1), negout.reshape(B, J, 1)


# speedup vs baseline: 2.4976x; 1.1612x over previous
import jax
import jax.numpy as jnp
from jax import lax
from jax.experimental import pallas as pl
from jax.experimental.pallas import tpu as pltpu
from jax.experimental.pallas import tpu_sc as plsc

NC = 2
NS = 16
NW = NC * NS
L = 16

B = 16384
J = 20
EMB = 64

B_W = B // NW
CB = 32
NCH = B_W // CB
NEG_ROWS = CB * J
G = 128
NG = NEG_ROWS // G
NB_W = B_W * J
PAD = L

V = 1000000
CPB = 8192
UPB = CPB // 2
NFB = V // CPB
TAIL0 = NFB * CPB
TAILN = V - TAIL0
VU = (NFB + 1) * UPB
CPB_LOG2 = CPB.bit_length() - 1
UPB_LOG2 = UPB.bit_length() - 1


def _tx_body(t_ref, out_ref):
    eye = jnp.eye(EMB, dtype=jnp.float32)
    x = t_ref[...]
    out_ref[:, 0:EMB] = jax.lax.dot_general(
        x[:, 0:UPB], eye, (((0,), (0,)), ((), ())),
        preferred_element_type=jnp.float32)
    out_ref[:, EMB:2 * EMB] = jax.lax.dot_general(
        x[:, UPB:CPB], eye, (((0,), (0,)), ((), ())),
        preferred_element_type=jnp.float32)


def _transpose_pack(table):
    return pl.pallas_call(
        _tx_body,
        grid=(NFB + 1,),
        in_specs=[pl.BlockSpec((EMB, CPB), lambda j: (0, j))],
        out_specs=pl.BlockSpec((UPB, 2 * EMB), lambda j: (j, 0)),
        out_shape=jax.ShapeDtypeStruct((VU, 2 * EMB), jnp.float32),
    )(table.T)


def _sgns_body(c_h, o_h, n_h, vemb, uemb, pos_h, negout_h,
               cidx, oidx, nidx, cg, og, ng, vrows, orows, nrows,
               posb, negb, sem):
    w = lax.axis_index("s") * NC + lax.axis_index("c")

    pltpu.sync_copy(c_h.at[w], cidx.at[pl.ds(0, B_W)])
    pltpu.sync_copy(o_h.at[w], oidx.at[pl.ds(0, B_W)])
    pltpu.sync_copy(n_h.at[w], nidx.at[pl.ds(0, NB_W)])

    lane = lax.iota(jnp.int32, L)
    last = lane == (L - 1)

    def unit_of(x):
        return lax.shift_left(lax.shift_right_logical(x, CPB_LOG2),
                              UPB_LOG2) | (x & (UPB - 1))

    def shift_co(i, carry):
        cg[pl.ds(i * L, L)] = unit_of(cidx[pl.ds(i * L, L)])
        og[pl.ds(i * L, L)] = unit_of(oidx[pl.ds(i * L, L)])
        return carry

    def shift_n(i, carry):
        ng[pl.ds(i * L, L)] = unit_of(nidx[pl.ds(i * L, L)])
        return carry

    lax.fori_loop(0, B_W // L, shift_co, 0)
    lax.fori_loop(0, NB_W // L, shift_n, 0)

    def chunk(ch, carry):
        cps = [
            pltpu.async_copy(vemb.at[cg.at[pl.ds(ch * CB, CB)]], vrows, sem),
            pltpu.async_copy(uemb.at[og.at[pl.ds(ch * CB, CB)]], orows, sem),
        ]
        for k in range(NG):
            cps.append(pltpu.async_copy(
                uemb.at[ng.at[pl.ds((ch * NG + k) * G, G)]],
                nrows.at[pl.ds(k * G, G)], sem))
        for cp in cps:
            cp.wait()

        def bbody(bl, c2):
            fb = ch * CB + bl
            co_vec = cidx[pl.ds(fb, L)]
            oo_vec = oidx[pl.ds(fb, L)]
            voff = jnp.where((co_vec[0] & UPB) != 0, EMB, 0)
            v0 = vrows[bl, pl.ds(voff, L)]
            v1 = vrows[bl, pl.ds(voff + L, L)]
            v2 = vrows[bl, pl.ds(voff + 2 * L, L)]
            v3 = vrows[bl, pl.ds(voff + 3 * L, L)]

            def dot_store(rref, row, off, pos):
                acc = rref[row, pl.ds(off, L)] * v0
                acc = acc + rref[row, pl.ds(off + L, L)] * v1
                acc = acc + rref[row, pl.ds(off + 2 * L, L)] * v2
                acc = acc + rref[row, pl.ds(off + 3 * L, L)] * v3
                s = plsc.cumsum(acc)
                idx = jnp.full((L,), pos, dtype=jnp.int32)
                plsc.store_scatter(negb, [idx], s, mask=last)

            po = jnp.where((oo_vec[0] & UPB) != 0, EMB, 0)
            acc = orows[bl, pl.ds(po, L)] * v0
            acc = acc + orows[bl, pl.ds(po + L, L)] * v1
            acc = acc + orows[bl, pl.ds(po + 2 * L, L)] * v2
            acc = acc + orows[bl, pl.ds(po + 3 * L, L)] * v3
            s = plsc.cumsum(acc)
            idx = jnp.full((L,), fb, dtype=jnp.int32)
            plsc.store_scatter(posb, [idx], s, mask=last)

            fnb = fb * J
            pv0 = nidx[pl.ds(fnb, L)]
            pv1 = nidx[pl.ds(fnb + 4, L)]
            for j in range(J):
                n_orig = pv0[j] if j < L else pv1[j - 4]
                dot_store(nrows, bl * J + j,
                          jnp.where((n_orig & UPB) != 0, EMB, 0), fnb + j)
            return c2

        lax.fori_loop(0, CB, bbody, 0)
        return carry

    lax.fori_loop(0, NCH, chunk, 0)

    def sig_pos(i, c2):
        x = posb[pl.ds(i * L, L)]
        posb[pl.ds(i * L, L)] = 1.0 / (1.0 + jnp.exp(-x))
        return c2

    def sig_neg(i, c2):
        x = negb[pl.ds(i * L, L)]
        negb[pl.ds(i * L, L)] = 1.0 / (1.0 + jnp.exp(x))
        return c2

    lax.fori_loop(0, B_W // L, sig_pos, 0)
    lax.fori_loop(0, NB_W // L, sig_neg, 0)

    pltpu.sync_copy(posb, pos_h.at[w])
    pltpu.sync_copy(negb, negout_h.at[w])


@jax.jit
def _sgns(c_h, o_h, n_h, vemb, uemb):
    mesh = plsc.VectorSubcoreMesh(core_axis_name="c", subcore_axis_name="s",
                                  num_cores=NC, num_subcores=NS)
    f = pl.kernel(
        _sgns_body,
        out_type=(
            jax.ShapeDtypeStruct((NW, B_W), jnp.float32),
            jax.ShapeDtypeStruct((NW, NB_W), jnp.float32),
        ),
        mesh=mesh,
        scratch_types=[
            pltpu.VMEM((B_W + PAD,), jnp.int32),
            pltpu.VMEM((B_W + PAD,), jnp.int32),
            pltpu.VMEM((NB_W + PAD,), jnp.int32),
            pltpu.VMEM((B_W,), jnp.int32),
            pltpu.VMEM((B_W,), jnp.int32),
            pltpu.VMEM((NB_W,), jnp.int32),
            pltpu.VMEM((CB, 2 * EMB), jnp.float32),
            pltpu.VMEM((CB, 2 * EMB), jnp.float32),
            pltpu.VMEM((NEG_ROWS, 2 * EMB), jnp.float32),
            pltpu.VMEM((B_W,), jnp.float32),
            pltpu.VMEM((NB_W,), jnp.float32),
            pltpu.SemaphoreType.DMA,
        ],
        compiler_params=pltpu.CompilerParams(needs_layout_passes=False,
                                             use_tc_tiling_on_sc=True),
    )
    return f(c_h, o_h, n_h, vemb, uemb)


def kernel(c, o, neg, vEmbedding, uEmbedding):
    c_h = c.reshape(NW, B_W).astype(jnp.int32)
    o_h = o.reshape(NW, B_W).astype(jnp.int32)
    n_h = neg.reshape(NW, NB_W).astype(jnp.int32)
    v2 = _transpose_pack(vEmbedding)
    u2 = _transpose_pack(uEmbedding)
    pos, negout = _sgns(c_h, o_h, n_h, v2, u2)
    return pos.reshape(B, 1), negout.reshape(B, J, 1)


# speedup vs baseline: 2.7127x; 1.0861x over previous
import jax
import jax.numpy as jnp
from jax import lax
from jax.experimental import pallas as pl
from jax.experimental.pallas import tpu as pltpu
from jax.experimental.pallas import tpu_sc as plsc

NC = 2
NS = 16
NW = NC * NS
L = 16

B = 16384
J = 20
EMB = 64

B_W = B // NW
CB = 32
NCH = B_W // CB
NEG_ROWS = CB * J
G = 128
NG = NEG_ROWS // G
NB_W = B_W * J
PAD = L

V = 1000000
CPB = 16384
UPB = CPB // 2
NFB = V // CPB
TAIL0 = NFB * CPB
TAILN = V - TAIL0
VU = (NFB + 1) * UPB
CPB_LOG2 = CPB.bit_length() - 1
UPB_LOG2 = UPB.bit_length() - 1


def _tx_body(t_ref, out_ref):
    eye = jnp.eye(EMB, dtype=jnp.float32)
    x = t_ref[...]
    out_ref[:, 0:EMB] = jax.lax.dot_general(
        x[:, 0:UPB], eye, (((0,), (0,)), ((), ())),
        preferred_element_type=jnp.float32)
    out_ref[:, EMB:2 * EMB] = jax.lax.dot_general(
        x[:, UPB:CPB], eye, (((0,), (0,)), ((), ())),
        preferred_element_type=jnp.float32)


def _transpose_pack(table):
    return pl.pallas_call(
        _tx_body,
        grid=(NFB + 1,),
        in_specs=[pl.BlockSpec((EMB, CPB), lambda j: (0, j))],
        out_specs=pl.BlockSpec((UPB, 2 * EMB), lambda j: (j, 0)),
        out_shape=jax.ShapeDtypeStruct((VU, 2 * EMB), jnp.float32),
    )(table.T)


def _sgns_body(c_h, o_h, n_h, vemb, uemb, pos_h, negout_h,
               cidx, oidx, nidx, cg, og, ng, vrows, orows, nrows,
               posb, negb, sem):
    w = lax.axis_index("s") * NC + lax.axis_index("c")

    pltpu.sync_copy(c_h.at[w], cidx.at[pl.ds(0, B_W)])
    pltpu.sync_copy(o_h.at[w], oidx.at[pl.ds(0, B_W)])
    pltpu.sync_copy(n_h.at[w], nidx.at[pl.ds(0, NB_W)])

    lane = lax.iota(jnp.int32, L)
    last = lane == (L - 1)

    def unit_of(x):
        return lax.shift_left(lax.shift_right_logical(x, CPB_LOG2),
                              UPB_LOG2) | (x & (UPB - 1))

    def shift_co(i, carry):
        cg[pl.ds(i * L, L)] = unit_of(cidx[pl.ds(i * L, L)])
        og[pl.ds(i * L, L)] = unit_of(oidx[pl.ds(i * L, L)])
        return carry

    def shift_n(i, carry):
        ng[pl.ds(i * L, L)] = unit_of(nidx[pl.ds(i * L, L)])
        return carry

    lax.fori_loop(0, B_W // L, shift_co, 0)
    lax.fori_loop(0, NB_W // L, shift_n, 0)

    def chunk(ch, carry):
        cps = [
            pltpu.async_copy(vemb.at[cg.at[pl.ds(ch * CB, CB)]], vrows, sem),
            pltpu.async_copy(uemb.at[og.at[pl.ds(ch * CB, CB)]], orows, sem),
        ]
        for k in range(NG):
            cps.append(pltpu.async_copy(
                uemb.at[ng.at[pl.ds((ch * NG + k) * G, G)]],
                nrows.at[pl.ds(k * G, G)], sem))
        for cp in cps:
            cp.wait()

        def bbody(bl, c2):
            fb = ch * CB + bl
            co_vec = cidx[pl.ds(fb, L)]
            oo_vec = oidx[pl.ds(fb, L)]
            voff = jnp.where((co_vec[0] & UPB) != 0, EMB, 0)
            v0 = vrows[bl, pl.ds(voff, L)]
            v1 = vrows[bl, pl.ds(voff + L, L)]
            v2 = vrows[bl, pl.ds(voff + 2 * L, L)]
            v3 = vrows[bl, pl.ds(voff + 3 * L, L)]

            def dot_store(rref, row, off, pos):
                acc = rref[row, pl.ds(off, L)] * v0
                acc = acc + rref[row, pl.ds(off + L, L)] * v1
                acc = acc + rref[row, pl.ds(off + 2 * L, L)] * v2
                acc = acc + rref[row, pl.ds(off + 3 * L, L)] * v3
                s = plsc.cumsum(acc)
                idx = jnp.full((L,), pos, dtype=jnp.int32)
                plsc.store_scatter(negb, [idx], s, mask=last)

            po = jnp.where((oo_vec[0] & UPB) != 0, EMB, 0)
            acc = orows[bl, pl.ds(po, L)] * v0
            acc = acc + orows[bl, pl.ds(po + L, L)] * v1
            acc = acc + orows[bl, pl.ds(po + 2 * L, L)] * v2
            acc = acc + orows[bl, pl.ds(po + 3 * L, L)] * v3
            s = plsc.cumsum(acc)
            idx = jnp.full((L,), fb, dtype=jnp.int32)
            plsc.store_scatter(posb, [idx], s, mask=last)

            fnb = fb * J
            pv0 = nidx[pl.ds(fnb, L)]
            pv1 = nidx[pl.ds(fnb + 4, L)]
            for j in range(J):
                n_orig = pv0[j] if j < L else pv1[j - 4]
                dot_store(nrows, bl * J + j,
                          jnp.where((n_orig & UPB) != 0, EMB, 0), fnb + j)
            return c2

        lax.fori_loop(0, CB, bbody, 0)
        return carry

    lax.fori_loop(0, NCH, chunk, 0)

    def sig_pos(i, c2):
        x = posb[pl.ds(i * L, L)]
        posb[pl.ds(i * L, L)] = 1.0 / (1.0 + jnp.exp(-x))
        return c2

    def sig_neg(i, c2):
        x = negb[pl.ds(i * L, L)]
        negb[pl.ds(i * L, L)] = 1.0 / (1.0 + jnp.exp(x))
        return c2

    lax.fori_loop(0, B_W // L, sig_pos, 0)
    lax.fori_loop(0, NB_W // L, sig_neg, 0)

    pltpu.sync_copy(posb, pos_h.at[w])
    pltpu.sync_copy(negb, negout_h.at[w])


@jax.jit
def _sgns(c_h, o_h, n_h, vemb, uemb):
    mesh = plsc.VectorSubcoreMesh(core_axis_name="c", subcore_axis_name="s",
                                  num_cores=NC, num_subcores=NS)
    f = pl.kernel(
        _sgns_body,
        out_type=(
            jax.ShapeDtypeStruct((NW, B_W), jnp.float32),
            jax.ShapeDtypeStruct((NW, NB_W), jnp.float32),
        ),
        mesh=mesh,
        scratch_types=[
            pltpu.VMEM((B_W + PAD,), jnp.int32),
            pltpu.VMEM((B_W + PAD,), jnp.int32),
            pltpu.VMEM((NB_W + PAD,), jnp.int32),
            pltpu.VMEM((B_W,), jnp.int32),
            pltpu.VMEM((B_W,), jnp.int32),
            pltpu.VMEM((NB_W,), jnp.int32),
            pltpu.VMEM((CB, 2 * EMB), jnp.float32),
            pltpu.VMEM((CB, 2 * EMB), jnp.float32),
            pltpu.VMEM((NEG_ROWS, 2 * EMB), jnp.float32),
            pltpu.VMEM((B_W,), jnp.float32),
            pltpu.VMEM((NB_W,), jnp.float32),
            pltpu.SemaphoreType.DMA,
        ],
        compiler_params=pltpu.CompilerParams(needs_layout_passes=False,
                                             use_tc_tiling_on_sc=True),
    )
    return f(c_h, o_h, n_h, vemb, uemb)


def kernel(c, o, neg, vEmbedding, uEmbedding):
    c_h = c.reshape(NW, B_W).astype(jnp.int32)
    o_h = o.reshape(NW, B_W).astype(jnp.int32)
    n_h = neg.reshape(NW, NB_W).astype(jnp.int32)
    v2 = _transpose_pack(vEmbedding)
    u2 = _transpose_pack(uEmbedding)
    pos, negout = _sgns(c_h, o_h, n_h, v2, u2)
    return pos.reshape(B, 1), negout.reshape(B, J, 1)


# speedup vs baseline: 2.8112x; 1.0363x over previous
import jax
import jax.numpy as jnp
from jax import lax
from jax.experimental import pallas as pl
from jax.experimental.pallas import tpu as pltpu
from jax.experimental.pallas import tpu_sc as plsc

NC = 2
NS = 16
NW = NC * NS
L = 16

B = 16384
J = 20
EMB = 64

B_W = B // NW
CB = 32
NCH = B_W // CB
NEG_ROWS = CB * J
G = 128
NG = NEG_ROWS // G
NB_W = B_W * J
PAD = L

V = 1000000
CPB = 32768
UPB = CPB // 2
NFB = V // CPB
TAIL0 = NFB * CPB
TAILN = V - TAIL0
VU = (NFB + 1) * UPB
CPB_LOG2 = CPB.bit_length() - 1
UPB_LOG2 = UPB.bit_length() - 1


def _tx_body(t_ref, out_ref):
    eye = jnp.eye(EMB, dtype=jnp.float32)
    x = t_ref[...]
    out_ref[:, 0:EMB] = jax.lax.dot_general(
        x[:, 0:UPB], eye, (((0,), (0,)), ((), ())),
        preferred_element_type=jnp.float32)
    out_ref[:, EMB:2 * EMB] = jax.lax.dot_general(
        x[:, UPB:CPB], eye, (((0,), (0,)), ((), ())),
        preferred_element_type=jnp.float32)


def _transpose_pack(table):
    return pl.pallas_call(
        _tx_body,
        grid=(NFB + 1,),
        in_specs=[pl.BlockSpec((EMB, CPB), lambda j: (0, j))],
        out_specs=pl.BlockSpec((UPB, 2 * EMB), lambda j: (j, 0)),
        out_shape=jax.ShapeDtypeStruct((VU, 2 * EMB), jnp.float32),
    )(table.T)


def _sgns_body(c_h, o_h, n_h, vemb, uemb, pos_h, negout_h,
               cidx, oidx, nidx, cg, og, ng, vrows, orows, nrows,
               posb, negb, sem):
    w = lax.axis_index("s") * NC + lax.axis_index("c")

    pltpu.sync_copy(c_h.at[w], cidx.at[pl.ds(0, B_W)])
    pltpu.sync_copy(o_h.at[w], oidx.at[pl.ds(0, B_W)])
    pltpu.sync_copy(n_h.at[w], nidx.at[pl.ds(0, NB_W)])

    lane = lax.iota(jnp.int32, L)
    last = lane == (L - 1)

    def unit_of(x):
        return lax.shift_left(lax.shift_right_logical(x, CPB_LOG2),
                              UPB_LOG2) | (x & (UPB - 1))

    def shift_co(i, carry):
        cg[pl.ds(i * L, L)] = unit_of(cidx[pl.ds(i * L, L)])
        og[pl.ds(i * L, L)] = unit_of(oidx[pl.ds(i * L, L)])
        return carry

    def shift_n(i, carry):
        ng[pl.ds(i * L, L)] = unit_of(nidx[pl.ds(i * L, L)])
        return carry

    lax.fori_loop(0, B_W // L, shift_co, 0)
    lax.fori_loop(0, NB_W // L, shift_n, 0)

    def chunk(ch, carry):
        cps = [
            pltpu.async_copy(vemb.at[cg.at[pl.ds(ch * CB, CB)]], vrows, sem),
            pltpu.async_copy(uemb.at[og.at[pl.ds(ch * CB, CB)]], orows, sem),
        ]
        for k in range(NG):
            cps.append(pltpu.async_copy(
                uemb.at[ng.at[pl.ds((ch * NG + k) * G, G)]],
                nrows.at[pl.ds(k * G, G)], sem))
        for cp in cps:
            cp.wait()

        def bbody(bl, c2):
            fb = ch * CB + bl
            co_vec = cidx[pl.ds(fb, L)]
            oo_vec = oidx[pl.ds(fb, L)]
            voff = jnp.where((co_vec[0] & UPB) != 0, EMB, 0)
            v0 = vrows[bl, pl.ds(voff, L)]
            v1 = vrows[bl, pl.ds(voff + L, L)]
            v2 = vrows[bl, pl.ds(voff + 2 * L, L)]
            v3 = vrows[bl, pl.ds(voff + 3 * L, L)]

            def dot_store(rref, row, off, pos):
                acc = rref[row, pl.ds(off, L)] * v0
                acc = acc + rref[row, pl.ds(off + L, L)] * v1
                acc = acc + rref[row, pl.ds(off + 2 * L, L)] * v2
                acc = acc + rref[row, pl.ds(off + 3 * L, L)] * v3
                s = plsc.cumsum(acc)
                idx = jnp.full((L,), pos, dtype=jnp.int32)
                plsc.store_scatter(negb, [idx], s, mask=last)

            po = jnp.where((oo_vec[0] & UPB) != 0, EMB, 0)
            acc = orows[bl, pl.ds(po, L)] * v0
            acc = acc + orows[bl, pl.ds(po + L, L)] * v1
            acc = acc + orows[bl, pl.ds(po + 2 * L, L)] * v2
            acc = acc + orows[bl, pl.ds(po + 3 * L, L)] * v3
            s = plsc.cumsum(acc)
            idx = jnp.full((L,), fb, dtype=jnp.int32)
            plsc.store_scatter(posb, [idx], s, mask=last)

            fnb = fb * J
            pv0 = nidx[pl.ds(fnb, L)]
            pv1 = nidx[pl.ds(fnb + 4, L)]
            for j in range(J):
                n_orig = pv0[j] if j < L else pv1[j - 4]
                dot_store(nrows, bl * J + j,
                          jnp.where((n_orig & UPB) != 0, EMB, 0), fnb + j)
            return c2

        lax.fori_loop(0, CB, bbody, 0)
        return carry

    lax.fori_loop(0, NCH, chunk, 0)

    def sig_pos(i, c2):
        x = posb[pl.ds(i * L, L)]
        posb[pl.ds(i * L, L)] = 1.0 / (1.0 + jnp.exp(-x))
        return c2

    def sig_neg(i, c2):
        x = negb[pl.ds(i * L, L)]
        negb[pl.ds(i * L, L)] = 1.0 / (1.0 + jnp.exp(x))
        return c2

    lax.fori_loop(0, B_W // L, sig_pos, 0)
    lax.fori_loop(0, NB_W // L, sig_neg, 0)

    pltpu.sync_copy(posb, pos_h.at[w])
    pltpu.sync_copy(negb, negout_h.at[w])


@jax.jit
def _sgns(c_h, o_h, n_h, vemb, uemb):
    mesh = plsc.VectorSubcoreMesh(core_axis_name="c", subcore_axis_name="s",
                                  num_cores=NC, num_subcores=NS)
    f = pl.kernel(
        _sgns_body,
        out_type=(
            jax.ShapeDtypeStruct((NW, B_W), jnp.float32),
            jax.ShapeDtypeStruct((NW, NB_W), jnp.float32),
        ),
        mesh=mesh,
        scratch_types=[
            pltpu.VMEM((B_W + PAD,), jnp.int32),
            pltpu.VMEM((B_W + PAD,), jnp.int32),
            pltpu.VMEM((NB_W + PAD,), jnp.int32),
            pltpu.VMEM((B_W,), jnp.int32),
            pltpu.VMEM((B_W,), jnp.int32),
            pltpu.VMEM((NB_W,), jnp.int32),
            pltpu.VMEM((CB, 2 * EMB), jnp.float32),
            pltpu.VMEM((CB, 2 * EMB), jnp.float32),
            pltpu.VMEM((NEG_ROWS, 2 * EMB), jnp.float32),
            pltpu.VMEM((B_W,), jnp.float32),
            pltpu.VMEM((NB_W,), jnp.float32),
            pltpu.SemaphoreType.DMA,
        ],
        compiler_params=pltpu.CompilerParams(needs_layout_passes=False,
                                             use_tc_tiling_on_sc=True),
    )
    return f(c_h, o_h, n_h, vemb, uemb)


def kernel(c, o, neg, vEmbedding, uEmbedding):
    c_h = c.reshape(NW, B_W).astype(jnp.int32)
    o_h = o.reshape(NW, B_W).astype(jnp.int32)
    n_h = neg.reshape(NW, NB_W).astype(jnp.int32)
    v2 = _transpose_pack(vEmbedding)
    u2 = _transpose_pack(uEmbedding)
    pos, negout = _sgns(c_h, o_h, n_h, v2, u2)
    return pos.reshape(B, 1), negout.reshape(B, J, 1)


# speedup vs baseline: 3.0903x; 1.0993x over previous
import jax
import jax.numpy as jnp
from jax import lax
from jax.experimental import pallas as pl
from jax.experimental.pallas import tpu as pltpu
from jax.experimental.pallas import tpu_sc as plsc

NC = 2
NS = 16
NW = NC * NS
L = 16

B = 16384
J = 20
EMB = 64

B_W = B // NW
CB = 16
NCH = B_W // CB
NEG_ROWS = CB * J
G = 64
NG = NEG_ROWS // G
NB_W = B_W * J
PAD = L

V = 1000000
CPB = 32768
UPB = CPB // 2
NFB = V // CPB
TAIL0 = NFB * CPB
TAILN = V - TAIL0
VU = (NFB + 1) * UPB
CPB_LOG2 = CPB.bit_length() - 1
UPB_LOG2 = UPB.bit_length() - 1


def _tx_body(t_ref, out_ref):
    eye = jnp.eye(EMB, dtype=jnp.float32)
    x = t_ref[...]
    out_ref[:, 0:EMB] = jax.lax.dot_general(
        x[:, 0:UPB], eye, (((0,), (0,)), ((), ())),
        preferred_element_type=jnp.float32)
    out_ref[:, EMB:2 * EMB] = jax.lax.dot_general(
        x[:, UPB:CPB], eye, (((0,), (0,)), ((), ())),
        preferred_element_type=jnp.float32)


def _transpose_pack(table):
    return pl.pallas_call(
        _tx_body,
        grid=(NFB + 1,),
        in_specs=[pl.BlockSpec((EMB, CPB), lambda j: (0, j))],
        out_specs=pl.BlockSpec((UPB, 2 * EMB), lambda j: (j, 0)),
        out_shape=jax.ShapeDtypeStruct((VU, 2 * EMB), jnp.float32),
    )(table.T)


def _sgns_body(c_h, o_h, n_h, vemb, uemb, pos_h, negout_h,
               cidx, oidx, nidx, cg, og, ng, vrows0, orows0, nrows0,
               vrows1, orows1, nrows1, posb, negb, sem0, sem1):
    w = lax.axis_index("s") * NC + lax.axis_index("c")

    pltpu.sync_copy(c_h.at[w], cidx.at[pl.ds(0, B_W)])
    pltpu.sync_copy(o_h.at[w], oidx.at[pl.ds(0, B_W)])
    pltpu.sync_copy(n_h.at[w], nidx.at[pl.ds(0, NB_W)])

    lane = lax.iota(jnp.int32, L)
    last = lane == (L - 1)

    def unit_of(x):
        return lax.shift_left(lax.shift_right_logical(x, CPB_LOG2),
                              UPB_LOG2) | (x & (UPB - 1))

    def shift_co(i, carry):
        cg[pl.ds(i * L, L)] = unit_of(cidx[pl.ds(i * L, L)])
        og[pl.ds(i * L, L)] = unit_of(oidx[pl.ds(i * L, L)])
        return carry

    def shift_n(i, carry):
        ng[pl.ds(i * L, L)] = unit_of(nidx[pl.ds(i * L, L)])
        return carry

    lax.fori_loop(0, B_W // L, shift_co, 0)
    lax.fori_loop(0, NB_W // L, shift_n, 0)

    bufs = ((vrows0, orows0, nrows0, sem0), (vrows1, orows1, nrows1, sem1))

    def gather_ops(ch, vrows, orows, nrows, sem):
        ops = [
            pltpu.make_async_copy(vemb.at[cg.at[pl.ds(ch * CB, CB)]],
                                  vrows, sem),
            pltpu.make_async_copy(uemb.at[og.at[pl.ds(ch * CB, CB)]],
                                  orows, sem),
        ]
        for k in range(NG):
            ops.append(pltpu.make_async_copy(
                uemb.at[ng.at[pl.ds((ch * NG + k) * G, G)]],
                nrows.at[pl.ds(k * G, G)], sem))
        return ops

    for op in gather_ops(0, *bufs[0]):
        op.start()

    def compute_chunk(ch, vrows, orows, nrows):
        def bbody(bl, c2):
            fb = ch * CB + bl
            co_vec = cidx[pl.ds(fb, L)]
            oo_vec = oidx[pl.ds(fb, L)]
            voff = jnp.where((co_vec[0] & UPB) != 0, EMB, 0)
            v0 = vrows[bl, pl.ds(voff, L)]
            v1 = vrows[bl, pl.ds(voff + L, L)]
            v2 = vrows[bl, pl.ds(voff + 2 * L, L)]
            v3 = vrows[bl, pl.ds(voff + 3 * L, L)]

            def dot_store(rref, row, off, pos):
                acc = rref[row, pl.ds(off, L)] * v0
                acc = acc + rref[row, pl.ds(off + L, L)] * v1
                acc = acc + rref[row, pl.ds(off + 2 * L, L)] * v2
                acc = acc + rref[row, pl.ds(off + 3 * L, L)] * v3
                s = plsc.cumsum(acc)
                idx = jnp.full((L,), pos, dtype=jnp.int32)
                plsc.store_scatter(negb, [idx], s, mask=last)

            po = jnp.where((oo_vec[0] & UPB) != 0, EMB, 0)
            acc = orows[bl, pl.ds(po, L)] * v0
            acc = acc + orows[bl, pl.ds(po + L, L)] * v1
            acc = acc + orows[bl, pl.ds(po + 2 * L, L)] * v2
            acc = acc + orows[bl, pl.ds(po + 3 * L, L)] * v3
            s = plsc.cumsum(acc)
            idx = jnp.full((L,), fb, dtype=jnp.int32)
            plsc.store_scatter(posb, [idx], s, mask=last)

            fnb = fb * J
            pv0 = nidx[pl.ds(fnb, L)]
            pv1 = nidx[pl.ds(fnb + 4, L)]
            for j in range(J):
                n_orig = pv0[j] if j < L else pv1[j - 4]
                dot_store(nrows, bl * J + j,
                          jnp.where((n_orig & UPB) != 0, EMB, 0), fnb + j)
            return c2

        lax.fori_loop(0, CB, bbody, 0)

    def chunk2(i, carry):
        for par in (0, 1):
            ch = i * 2 + par

            @pl.when(ch + 1 < NCH)
            def _prefetch(par=par, ch=ch):
                for op in gather_ops(ch + 1, *bufs[1 - par]):
                    op.start()

            vrows, orows, nrows, sem = bufs[par]
            for op in gather_ops(ch, vrows, orows, nrows, sem):
                op.wait()
            compute_chunk(ch, vrows, orows, nrows)
        return carry

    lax.fori_loop(0, NCH // 2, chunk2, 0)

    def sig_pos(i, c2):
        x = posb[pl.ds(i * L, L)]
        posb[pl.ds(i * L, L)] = 1.0 / (1.0 + jnp.exp(-x))
        return c2

    def sig_neg(i, c2):
        x = negb[pl.ds(i * L, L)]
        negb[pl.ds(i * L, L)] = 1.0 / (1.0 + jnp.exp(x))
        return c2

    lax.fori_loop(0, B_W // L, sig_pos, 0)
    lax.fori_loop(0, NB_W // L, sig_neg, 0)

    pltpu.sync_copy(posb, pos_h.at[w])
    pltpu.sync_copy(negb, negout_h.at[w])


@jax.jit
def _sgns(c_h, o_h, n_h, vemb, uemb):
    mesh = plsc.VectorSubcoreMesh(core_axis_name="c", subcore_axis_name="s",
                                  num_cores=NC, num_subcores=NS)
    f = pl.kernel(
        _sgns_body,
        out_type=(
            jax.ShapeDtypeStruct((NW, B_W), jnp.float32),
            jax.ShapeDtypeStruct((NW, NB_W), jnp.float32),
        ),
        mesh=mesh,
        scratch_types=[
            pltpu.VMEM((B_W + PAD,), jnp.int32),
            pltpu.VMEM((B_W + PAD,), jnp.int32),
            pltpu.VMEM((NB_W + PAD,), jnp.int32),
            pltpu.VMEM((B_W,), jnp.int32),
            pltpu.VMEM((B_W,), jnp.int32),
            pltpu.VMEM((NB_W,), jnp.int32),
            pltpu.VMEM((CB, 2 * EMB), jnp.float32),
            pltpu.VMEM((CB, 2 * EMB), jnp.float32),
            pltpu.VMEM((NEG_ROWS, 2 * EMB), jnp.float32),
            pltpu.VMEM((CB, 2 * EMB), jnp.float32),
            pltpu.VMEM((CB, 2 * EMB), jnp.float32),
            pltpu.VMEM((NEG_ROWS, 2 * EMB), jnp.float32),
            pltpu.VMEM((B_W,), jnp.float32),
            pltpu.VMEM((NB_W,), jnp.float32),
            pltpu.SemaphoreType.DMA,
            pltpu.SemaphoreType.DMA,
        ],
        compiler_params=pltpu.CompilerParams(needs_layout_passes=False,
                                             use_tc_tiling_on_sc=True),
    )
    return f(c_h, o_h, n_h, vemb, uemb)


def kernel(c, o, neg, vEmbedding, uEmbedding):
    c_h = c.reshape(NW, B_W).astype(jnp.int32)
    o_h = o.reshape(NW, B_W).astype(jnp.int32)
    n_h = neg.reshape(NW, NB_W).astype(jnp.int32)
    v2 = _transpose_pack(vEmbedding)
    u2 = _transpose_pack(uEmbedding)
    pos, negout = _sgns(c_h, o_h, n_h, v2, u2)
    return pos.reshape(B, 1), negout.reshape(B, J, 1)
